# baseline (speedup 1.0000x reference)
import dataclasses
import functools

import jax
import jax.numpy as jnp
from jax import lax
from jax.experimental import pallas as pl
from jax.experimental.pallas import tpu as pltpu
from jax.experimental.pallas import tpu_sc as plsc

N = 10000
E = 320000
D = 128

NPAD = 10240
NW = 32
RPW = NPAD // NW
TRASH = RPW
CHUNK = 4000
NCHUNK = E // CHUNK
GB = 192
NGMAX = GB // 16
W32 = D // 2
LCAP = CHUNK + GB + 32
BN_SCALE = float(1.0 / (1.0 + 1e-5) ** 0.5)


def _dense_pre_body(x_ref, wp_ref, bp_ref, ws_ref, m_ref, hws_ref):
    h = jnp.log(x_ref[...] + 1.0)
    m = jax.nn.relu(
        jnp.dot(h, wp_ref[...], preferred_element_type=jnp.float32) + bp_ref[...]
    )
    m_ref[...] = m.astype(jnp.bfloat16)
    hws_ref[...] = jnp.dot(h, ws_ref[...], preferred_element_type=jnp.float32)


def _dense_pre(xp, W_pool, b_pool2d, W_self):
    blk = NPAD // 8
    return pl.pallas_call(
        _dense_pre_body,
        grid=(8,),
        in_specs=[
            pl.BlockSpec((blk, D), lambda i: (i, 0)),
            pl.BlockSpec((D, D), lambda i: (0, 0)),
            pl.BlockSpec((1, D), lambda i: (0, 0)),
            pl.BlockSpec((D, D), lambda i: (0, 0)),
        ],
        out_specs=[
            pl.BlockSpec((blk, D), lambda i: (i, 0)),
            pl.BlockSpec((blk, D), lambda i: (i, 0)),
        ],
        out_shape=[
            jax.ShapeDtypeStruct((NPAD, D), jnp.bfloat16),
            jax.ShapeDtypeStruct((NPAD, D), jnp.float32),
        ],
    )(xp, W_pool, b_pool2d, W_self)


def _seg_max_body(src_hbm, dst_hbm, m_hbm, agg_hbm,
                  srcv0, srcv1, dstv0, dstv1, sl0, sl1, dl0, dl1,
                  grows0, grows1, acc, se0, se1, sg0, sg1):
    srcv = (srcv0, srcv1)
    dstv = (dstv0, dstv1)
    slist = (sl0, sl1)
    dlist = (dl0, dl1)
    grows = (grows0, grows1)
    esems = (se0, se1)
    gsems = (sg0, sg1)

    wid = lax.axis_index("s") * 2 + lax.axis_index("c")
    lo = wid * RPW
    lanes = lax.broadcasted_iota(jnp.int32, (16,), 0)
    zero16 = jnp.zeros((16,), jnp.int32)
    trash16 = jnp.full((16,), TRASH, jnp.int32)

    def estart(par, ci):
        off = ci * CHUNK
        pltpu.make_async_copy(
            src_hbm.at[pl.ds(off, CHUNK)], srcv[par], esems[par]).start()
        pltpu.make_async_copy(
            dst_hbm.at[pl.ds(off, CHUNK)], dstv[par], esems[par]).start()

    def ewait(par):
        pltpu.make_async_copy(
            src_hbm.at[pl.ds(0, CHUNK)], srcv[par], esems[par]).wait()
        pltpu.make_async_copy(
            dst_hbm.at[pl.ds(0, CHUNK)], dstv[par], esems[par]).wait()

    def gstart(buf, listpar, b):
        pltpu.make_async_copy(
            m_hbm.at[slist[listpar].at[pl.ds(b * GB, GB)]], grows[buf],
            gsems[buf]).start()

    def gwait(buf):
        pltpu.make_async_copy(
            m_hbm.at[slist[0].at[pl.ds(0, GB)]], grows[buf],
            gsems[buf]).wait()

    @pl.loop(0, RPW + 1)
    def _(r):
        @pl.loop(0, W32, step=16)
        def _(k):
            acc[r, pl.ds(k, 16)] = zero16

    for p in range(2):
        @pl.loop(0, LCAP, step=16)
        def _(i):
            slist[p][pl.ds(i, 16)] = zero16

    def accum(buf, listpar, goff, gcnt):
        def group_body(g, carry):
            dlv = dlist[listpar][pl.ds((goff + g) * 16, 16)]
            for lane in range(16):
                dl = dlv[lane]
                e = g * 16 + lane
                for k in range(W32 // 16):
                    cur = plsc.bitcast(acc[dl, pl.ds(k * 16, 16)], jnp.bfloat16)
                    new = plsc.bitcast(grows[buf][e, pl.ds(k * 16, 16)],
                                       jnp.bfloat16)
                    acc[dl, pl.ds(k * 16, 16)] = plsc.bitcast(
                        jnp.maximum(cur, new), jnp.int32)
            return carry

        lax.fori_loop(0, gcnt, group_body, jnp.int32(0))

    def scan(par):
        def scan_body(i, ptr):
            base = i * 32
            d0 = dstv[par][pl.ds(base, 16)]
            s0 = srcv[par][pl.ds(base, 16)]
            d1 = dstv[par][pl.ds(base + 16, 16)]
            s1 = srcv[par][pl.ds(base + 16, 16)]
            r0 = d0 - lo
            r1 = d1 - lo
            m0 = r0.astype(jnp.uint32) < jnp.uint32(RPW)
            m1 = r1.astype(jnp.uint32) < jnp.uint32(RPW)
            plsc.store_compressed(slist[par].at[pl.ds(ptr, 16)], s0, mask=m0)
            plsc.store_compressed(dlist[par].at[pl.ds(ptr, 16)], r0, mask=m0)
            p1 = ptr + plsc.all_reduce_population_count(m0)[0]
            plsc.store_compressed(slist[par].at[pl.ds(p1, 16)], s1, mask=m1)
            plsc.store_compressed(dlist[par].at[pl.ds(p1, 16)], r1, mask=m1)
            return p1 + plsc.all_reduce_population_count(m1)[0]

        return lax.fori_loop(0, CHUNK // 32, scan_body, jnp.int32(0))

    def section(par, ci2):
        cur = ci2 * 2 + par

        def run(prev_ng):
            nxt = cur + 1
            estart(1 - par, jnp.where(nxt < NCHUNK, nxt, 0))
            ewait(par)
            ptr = scan(par)
            slist[par][pl.ds(ptr, 16)] = lanes
            dlist[par][pl.ds(ptr, 16)] = trash16
            gstart(par, par, 0)
            gwait(1 - par)
            accum(1 - par, 1 - par, jnp.int32(0), prev_ng)
            ng_all = (ptr + 15) // 16
            nxb = (ng_all + NGMAX - 1) // NGMAX

            @pl.when(nxb > 1)
            def _():
                def extra(b, carry):
                    gstart(1 - par, par, b)
                    gwait(1 - par)
                    accum(1 - par, par, b * NGMAX,
                          jnp.minimum(ng_all - b * NGMAX, NGMAX))
                    return carry
                lax.fori_loop(1, nxb, extra, jnp.int32(0))

            return jnp.minimum(ng_all, NGMAX)
        return run

    def chunk_pair(ci2, prev_ng):
        prev_ng = section(0, ci2)(prev_ng)
        prev_ng = section(1, ci2)(prev_ng)
        return prev_ng

    estart(0, jnp.int32(0))
    gstart(1, 0, 0)
    prev_ng = lax.fori_loop(0, NCHUNK // 2, chunk_pair, jnp.int32(0))

    gwait(1)
    accum(1, 1, jnp.int32(0), prev_ng)
    ewait(0)

    pltpu.sync_copy(acc.at[pl.ds(0, RPW)], agg_hbm.at[pl.ds(lo, RPW)])


def _seg_max(src, dst, m):
    mesh = plsc.VectorSubcoreMesh(core_axis_name="c", subcore_axis_name="s")
    cp = pltpu.CompilerParams()
    if "needs_layout_passes" in pltpu.CompilerParams.__dataclass_fields__:
        cp = dataclasses.replace(cp, needs_layout_passes=False)
    f = pl.kernel(
        _seg_max_body,
        out_type=jax.ShapeDtypeStruct((NPAD, W32), jnp.int32),
        mesh=mesh,
        compiler_params=cp,
        scratch_types=[
            pltpu.VMEM((CHUNK,), jnp.int32),
            pltpu.VMEM((CHUNK,), jnp.int32),
            pltpu.VMEM((CHUNK,), jnp.int32),
            pltpu.VMEM((CHUNK,), jnp.int32),
            pltpu.VMEM((LCAP,), jnp.int32),
            pltpu.VMEM((LCAP,), jnp.int32),
            pltpu.VMEM((LCAP,), jnp.int32),
            pltpu.VMEM((LCAP,), jnp.int32),
            pltpu.VMEM((GB, D), jnp.int32),
            pltpu.VMEM((GB, D), jnp.int32),
            pltpu.VMEM((RPW + 1, W32), jnp.int32),
            pltpu.SemaphoreType.DMA,
            pltpu.SemaphoreType.DMA,
            pltpu.SemaphoreType.DMA,
            pltpu.SemaphoreType.DMA,
        ],
    )
    return f(src, dst, m)


def _dense_post_body(hws_ref, agg_ref, wn_ref, bs_ref, wf_ref, bf_ref,
                     g_ref, be_ref, z_ref):
    agg = agg_ref[...].astype(jnp.float32)
    rst = jax.nn.relu(
        hws_ref[...]
        + jnp.dot(agg, wn_ref[...], preferred_element_type=jnp.float32)
        + bs_ref[...]
    )
    nrm = jnp.maximum(
        jnp.sqrt(jnp.sum(rst * rst, axis=1, keepdims=True)), 1e-12)
    rst = rst / nrm
    z = jnp.dot(rst, wf_ref[...], preferred_element_type=jnp.float32) + bf_ref[...]
    z = z * (BN_SCALE * g_ref[...]) + be_ref[...]
    z_ref[...] = jax.nn.relu(z)


def _dense_post(hws, agg, W_neigh, b_sage2d, W_fc, b_fc2d, gamma2d, beta2d):
    blk = NPAD // 8
    return pl.pallas_call(
        _dense_post_body,
        grid=(8,),
        in_specs=[
            pl.BlockSpec((blk, D), lambda i: (i, 0)),
            pl.BlockSpec((blk, D), lambda i: (i, 0)),
            pl.BlockSpec((D, D), lambda i: (0, 0)),
            pl.BlockSpec((1, D), lambda i: (0, 0)),
            pl.BlockSpec((D, D), lambda i: (0, 0)),
            pl.BlockSpec((1, D), lambda i: (0, 0)),
            pl.BlockSpec((1, D), lambda i: (0, 0)),
            pl.BlockSpec((1, D), lambda i: (0, 0)),
        ],
        out_specs=pl.BlockSpec((blk, D), lambda i: (i, 0)),
        out_shape=jax.ShapeDtypeStruct((NPAD, D), jnp.float32),
    )(hws, agg, W_neigh, b_sage2d, W_fc, b_fc2d, gamma2d, beta2d)


def kernel(x, edge_index, W_pool, b_pool, W_self, W_neigh, b_sage,
           W_fc, b_fc, gamma, beta):
    xp = jnp.zeros((NPAD, D), jnp.float32).at[:N].set(x)
    m, hws = _dense_pre(xp, W_pool, b_pool.reshape(1, D), W_self)
    m32 = lax.bitcast_convert_type(m.reshape(NPAD, W32, 2), jnp.int32)
    m32 = jnp.pad(m32, ((0, 0), (0, D - W32)))
    agg32 = _seg_max(edge_index[0], edge_index[1], m32)
    agg = lax.bitcast_convert_type(agg32, jnp.bfloat16).reshape(NPAD, D)
    z = _dense_post(hws, agg, W_neigh, b_sage.reshape(1, D), W_fc,
                    b_fc.reshape(1, D), gamma.reshape(1, D), beta.reshape(1, D))
    return z[:N]


# speedup vs baseline: 6.7787x; 6.7787x over previous
import dataclasses
import functools

import jax
import jax.numpy as jnp
from jax import lax
from jax.experimental import pallas as pl
from jax.experimental.pallas import tpu as pltpu
from jax.experimental.pallas import tpu_sc as plsc

N = 10000
E = 320000
D = 128

NPAD = 10240
NW = 32
RPW = NPAD // NW
TRASH = RPW
CHUNK = 4000
NCHUNK = E // CHUNK
GB = 192
NGMAX = GB // 16
SUB = 48
NSUB = GB // SUB
GPS = SUB // 16
W32 = D // 2
LCAP = CHUNK + GB + 32
BN_SCALE = float(1.0 / (1.0 + 1e-5) ** 0.5)


def _dense_pre_body(x_ref, wp_ref, bp_ref, ws_ref, m_ref, hws_ref):
    h = jnp.log(x_ref[...] + 1.0)
    m = jax.nn.relu(
        jnp.dot(h, wp_ref[...], preferred_element_type=jnp.float32) + bp_ref[...]
    )
    m_ref[...] = m.astype(jnp.bfloat16)
    hws_ref[...] = jnp.dot(h, ws_ref[...], preferred_element_type=jnp.float32)


def _dense_pre(xp, W_pool, b_pool2d, W_self):
    blk = NPAD // 8
    return pl.pallas_call(
        _dense_pre_body,
        grid=(8,),
        in_specs=[
            pl.BlockSpec((blk, D), lambda i: (i, 0)),
            pl.BlockSpec((D, D), lambda i: (0, 0)),
            pl.BlockSpec((1, D), lambda i: (0, 0)),
            pl.BlockSpec((D, D), lambda i: (0, 0)),
        ],
        out_specs=[
            pl.BlockSpec((blk, D), lambda i: (i, 0)),
            pl.BlockSpec((blk, D), lambda i: (i, 0)),
        ],
        out_shape=[
            jax.ShapeDtypeStruct((NPAD, D), jnp.bfloat16),
            jax.ShapeDtypeStruct((NPAD, D), jnp.float32),
        ],
    )(xp, W_pool, b_pool2d, W_self)


def _seg_max_body(src_hbm, dst_hbm, m_hbm, agg_hbm,
                  srcv0, srcv1, dstv0, dstv1, sl0, sl1, dl0, dl1,
                  grows0, grows1, acc, se0, se1,
                  sg00, sg01, sg02, sg03, sg10, sg11, sg12, sg13):
    srcv = (srcv0, srcv1)
    dstv = (dstv0, dstv1)
    slist = (sl0, sl1)
    dlist = (dl0, dl1)
    grows = (grows0, grows1)
    esems = (se0, se1)
    gsems = ((sg00, sg01, sg02, sg03), (sg10, sg11, sg12, sg13))

    wid = lax.axis_index("s") * 2 + lax.axis_index("c")
    lo = wid * RPW
    lanes = lax.broadcasted_iota(jnp.int32, (16,), 0)
    zero16 = jnp.zeros((16,), jnp.int32)
    trash16 = jnp.full((16,), TRASH, jnp.int32)

    def estart(par, ci):
        off = ci * CHUNK
        pltpu.make_async_copy(
            src_hbm.at[pl.ds(off, CHUNK)], srcv[par], esems[par]).start()
        pltpu.make_async_copy(
            dst_hbm.at[pl.ds(off, CHUNK)], dstv[par], esems[par]).start()

    def ewait(par):
        pltpu.make_async_copy(
            src_hbm.at[pl.ds(0, CHUNK)], srcv[par], esems[par]).wait()
        pltpu.make_async_copy(
            dst_hbm.at[pl.ds(0, CHUNK)], dstv[par], esems[par]).wait()

    def gstart(buf, listpar, b, nsub):
        for i in range(NSUB):
            @pl.when(i < nsub)
            def _():
                pltpu.make_async_copy(
                    m_hbm.at[slist[listpar].at[pl.ds(b * GB + i * SUB, SUB)]],
                    grows[buf].at[pl.ds(i * SUB, SUB)],
                    gsems[buf][i]).start()

    def gwait(buf, nsub):
        for i in range(NSUB):
            @pl.when(i < nsub)
            def _():
                pltpu.make_async_copy(
                    m_hbm.at[slist[0].at[pl.ds(0, SUB)]],
                    grows[buf].at[pl.ds(0, SUB)],
                    gsems[buf][i]).wait()

    @pl.loop(0, RPW + 1)
    def _(r):
        @pl.loop(0, W32, step=16)
        def _(k):
            acc[r, pl.ds(k, 16)] = zero16

    for p in range(2):
        @pl.loop(0, LCAP, step=16)
        def _(i):
            slist[p][pl.ds(i, 16)] = zero16

    def accum(buf, listpar, goff, gcnt):
        def group_body(g, carry):
            dlv = dlist[listpar][pl.ds((goff + g) * 16, 16)]
            for lane in range(16):
                dl = dlv[lane]
                e = g * 16 + lane
                for k in range(W32 // 16):
                    cur = plsc.bitcast(acc[dl, pl.ds(k * 16, 16)], jnp.bfloat16)
                    new = plsc.bitcast(grows[buf][e, pl.ds(k * 16, 16)],
                                       jnp.bfloat16)
                    acc[dl, pl.ds(k * 16, 16)] = plsc.bitcast(
                        jnp.maximum(cur, new), jnp.int32)
            return carry

        lax.fori_loop(0, gcnt, group_body, jnp.int32(0))

    def scan(par):
        def scan_body(i, ptr):
            base = i * 32
            d0 = dstv[par][pl.ds(base, 16)]
            s0 = srcv[par][pl.ds(base, 16)]
            d1 = dstv[par][pl.ds(base + 16, 16)]
            s1 = srcv[par][pl.ds(base + 16, 16)]
            r0 = d0 - lo
            r1 = d1 - lo
            m0 = r0.astype(jnp.uint32) < jnp.uint32(RPW)
            m1 = r1.astype(jnp.uint32) < jnp.uint32(RPW)
            plsc.store_compressed(slist[par].at[pl.ds(ptr, 16)], s0, mask=m0)
            plsc.store_compressed(dlist[par].at[pl.ds(ptr, 16)], r0, mask=m0)
            p1 = ptr + plsc.all_reduce_population_count(m0)[0]
            plsc.store_compressed(slist[par].at[pl.ds(p1, 16)], s1, mask=m1)
            plsc.store_compressed(dlist[par].at[pl.ds(p1, 16)], r1, mask=m1)
            return p1 + plsc.all_reduce_population_count(m1)[0]

        return lax.fori_loop(0, CHUNK // 32, scan_body, jnp.int32(0))

    def section(par, ci2):
        cur = ci2 * 2 + par

        def run(carry):
            prev_ng, prev_nsub = carry
            nxt = cur + 1
            estart(1 - par, jnp.where(nxt < NCHUNK, nxt, 0))
            ewait(par)
            ptr = scan(par)
            slist[par][pl.ds(ptr, 16)] = lanes
            dlist[par][pl.ds(ptr, 16)] = trash16
            ng_all = (ptr + 15) // 16
            ng = jnp.minimum(ng_all, NGMAX)
            nsub = (ng + (GPS - 1)) // GPS
            gstart(par, par, 0, nsub)
            gwait(1 - par, prev_nsub)
            accum(1 - par, 1 - par, jnp.int32(0), prev_ng)
            nxb = (ng_all + NGMAX - 1) // NGMAX

            @pl.when(nxb > 1)
            def _():
                def extra(b, carry2):
                    gstart(1 - par, par, b, jnp.int32(NSUB))
                    gwait(1 - par, jnp.int32(NSUB))
                    accum(1 - par, par, b * NGMAX,
                          jnp.minimum(ng_all - b * NGMAX, NGMAX))
                    return carry2
                lax.fori_loop(1, nxb, extra, jnp.int32(0))

            return (ng, nsub)
        return run

    def chunk_pair(ci2, carry):
        carry = section(0, ci2)(carry)
        carry = section(1, ci2)(carry)
        return carry

    estart(0, jnp.int32(0))
    carry = lax.fori_loop(0, NCHUNK // 2, chunk_pair,
                          (jnp.int32(0), jnp.int32(0)))
    prev_ng, prev_nsub = carry

    gwait(1, prev_nsub)
    accum(1, 1, jnp.int32(0), prev_ng)
    ewait(0)

    pltpu.sync_copy(acc.at[pl.ds(0, RPW)], agg_hbm.at[pl.ds(lo, RPW)])


def _seg_max(src, dst, m):
    mesh = plsc.VectorSubcoreMesh(core_axis_name="c", subcore_axis_name="s")
    cp = pltpu.CompilerParams()
    if "needs_layout_passes" in pltpu.CompilerParams.__dataclass_fields__:
        cp = dataclasses.replace(cp, needs_layout_passes=False)
    f = pl.kernel(
        _seg_max_body,
        out_type=jax.ShapeDtypeStruct((NPAD, W32), jnp.int32),
        mesh=mesh,
        compiler_params=cp,
        scratch_types=[
            pltpu.VMEM((CHUNK,), jnp.int32),
            pltpu.VMEM((CHUNK,), jnp.int32),
            pltpu.VMEM((CHUNK,), jnp.int32),
            pltpu.VMEM((CHUNK,), jnp.int32),
            pltpu.VMEM((LCAP,), jnp.int32),
            pltpu.VMEM((LCAP,), jnp.int32),
            pltpu.VMEM((LCAP,), jnp.int32),
            pltpu.VMEM((LCAP,), jnp.int32),
            pltpu.VMEM((GB, D), jnp.int32),
            pltpu.VMEM((GB, D), jnp.int32),
            pltpu.VMEM((RPW + 1, W32), jnp.int32),
        ] + [pltpu.SemaphoreType.DMA] * (2 + 2 * NSUB),
    )
    return f(src, dst, m)


def _dense_post_body(hws_ref, agg_ref, wn_ref, bs_ref, wf_ref, bf_ref,
                     g_ref, be_ref, z_ref):
    agg = agg_ref[...].astype(jnp.float32)
    rst = jax.nn.relu(
        hws_ref[...]
        + jnp.dot(agg, wn_ref[...], preferred_element_type=jnp.float32)
        + bs_ref[...]
    )
    nrm = jnp.maximum(
        jnp.sqrt(jnp.sum(rst * rst, axis=1, keepdims=True)), 1e-12)
    rst = rst / nrm
    z = jnp.dot(rst, wf_ref[...], preferred_element_type=jnp.float32) + bf_ref[...]
    z = z * (BN_SCALE * g_ref[...]) + be_ref[...]
    z_ref[...] = jax.nn.relu(z)


def _dense_post(hws, agg, W_neigh, b_sage2d, W_fc, b_fc2d, gamma2d, beta2d):
    blk = NPAD // 8
    return pl.pallas_call(
        _dense_post_body,
        grid=(8,),
        in_specs=[
            pl.BlockSpec((blk, D), lambda i: (i, 0)),
            pl.BlockSpec((blk, D), lambda i: (i, 0)),
            pl.BlockSpec((D, D), lambda i: (0, 0)),
            pl.BlockSpec((1, D), lambda i: (0, 0)),
            pl.BlockSpec((D, D), lambda i: (0, 0)),
            pl.BlockSpec((1, D), lambda i: (0, 0)),
            pl.BlockSpec((1, D), lambda i: (0, 0)),
            pl.BlockSpec((1, D), lambda i: (0, 0)),
        ],
        out_specs=pl.BlockSpec((blk, D), lambda i: (i, 0)),
        out_shape=jax.ShapeDtypeStruct((NPAD, D), jnp.float32),
    )(hws, agg, W_neigh, b_sage2d, W_fc, b_fc2d, gamma2d, beta2d)


def kernel(x, edge_index, W_pool, b_pool, W_self, W_neigh, b_sage,
           W_fc, b_fc, gamma, beta):
    xp = jnp.zeros((NPAD, D), jnp.float32).at[:N].set(x)
    m, hws = _dense_pre(xp, W_pool, b_pool.reshape(1, D), W_self)
    m32 = lax.bitcast_convert_type(m.reshape(NPAD, W32, 2), jnp.int32)
    m32 = jnp.pad(m32, ((0, 0), (0, D - W32)))
    agg32 = _seg_max(edge_index[0], edge_index[1], m32)
    agg = lax.bitcast_convert_type(agg32, jnp.bfloat16).reshape(NPAD, D)
    z = _dense_post(hws, agg, W_neigh, b_sage.reshape(1, D), W_fc,
                    b_fc.reshape(1, D), gamma.reshape(1, D), beta.reshape(1, D))
    return z[:N]


# speedup vs baseline: 7.1129x; 1.0493x over previous
import dataclasses
import functools

import jax
import jax.numpy as jnp
from jax import lax
from jax.experimental import pallas as pl
from jax.experimental.pallas import tpu as pltpu
from jax.experimental.pallas import tpu_sc as plsc

N = 10000
E = 320000
D = 128

NPAD = 10240
NW = 32
RPW = NPAD // NW
TRASH = RPW
CHUNK = 4000
NCHUNK = E // CHUNK
GB = 192
NGMAX = GB // 16
SUB = 32
NSUB = GB // SUB
GPS = SUB // 16
W32 = D // 2
LCAP = CHUNK + GB + 32
BN_SCALE = float(1.0 / (1.0 + 1e-5) ** 0.5)


def _dense_pre_body(x_ref, wp_ref, bp_ref, ws_ref, m_ref, hws_ref):
    h = jnp.log(x_ref[...] + 1.0)
    m = jax.nn.relu(
        jnp.dot(h, wp_ref[...], preferred_element_type=jnp.float32) + bp_ref[...]
    )
    m_ref[...] = m.astype(jnp.bfloat16)
    hws_ref[...] = jnp.dot(h, ws_ref[...], preferred_element_type=jnp.float32)


def _dense_pre(xp, W_pool, b_pool2d, W_self):
    blk = NPAD // 8
    return pl.pallas_call(
        _dense_pre_body,
        grid=(8,),
        in_specs=[
            pl.BlockSpec((blk, D), lambda i: (i, 0)),
            pl.BlockSpec((D, D), lambda i: (0, 0)),
            pl.BlockSpec((1, D), lambda i: (0, 0)),
            pl.BlockSpec((D, D), lambda i: (0, 0)),
        ],
        out_specs=[
            pl.BlockSpec((blk, D), lambda i: (i, 0)),
            pl.BlockSpec((blk, D), lambda i: (i, 0)),
        ],
        out_shape=[
            jax.ShapeDtypeStruct((NPAD, D), jnp.bfloat16),
            jax.ShapeDtypeStruct((NPAD, D), jnp.float32),
        ],
    )(xp, W_pool, b_pool2d, W_self)


def _seg_max_body(src_hbm, dst_hbm, m_hbm, agg_hbm,
                  srcv0, srcv1, dstv0, dstv1, sl0, sl1, dl0, dl1,
                  grows0, grows1, acc, se0, se1,
                  sg00, sg01, sg02, sg03, sg04, sg05,
                  sg10, sg11, sg12, sg13, sg14, sg15):
    srcv = (srcv0, srcv1)
    dstv = (dstv0, dstv1)
    slist = (sl0, sl1)
    dlist = (dl0, dl1)
    grows = (grows0, grows1)
    esems = (se0, se1)
    gsems = ((sg00, sg01, sg02, sg03, sg04, sg05),
             (sg10, sg11, sg12, sg13, sg14, sg15))

    wid = lax.axis_index("s") * 2 + lax.axis_index("c")
    lo = wid * RPW
    lanes = lax.broadcasted_iota(jnp.int32, (16,), 0)
    zero16 = jnp.zeros((16,), jnp.int32)
    trash16 = jnp.full((16,), TRASH, jnp.int32)

    def estart(par, ci):
        off = ci * CHUNK
        pltpu.make_async_copy(
            src_hbm.at[pl.ds(off, CHUNK)], srcv[par], esems[par]).start()
        pltpu.make_async_copy(
            dst_hbm.at[pl.ds(off, CHUNK)], dstv[par], esems[par]).start()

    def ewait(par):
        pltpu.make_async_copy(
            src_hbm.at[pl.ds(0, CHUNK)], srcv[par], esems[par]).wait()
        pltpu.make_async_copy(
            dst_hbm.at[pl.ds(0, CHUNK)], dstv[par], esems[par]).wait()

    def gstart(buf, listpar, b, nsub):
        for i in range(NSUB):
            @pl.when(i < nsub)
            def _():
                pltpu.make_async_copy(
                    m_hbm.at[slist[listpar].at[pl.ds(b * GB + i * SUB, SUB)]],
                    grows[buf].at[pl.ds(i * SUB, SUB)],
                    gsems[buf][i]).start()

    def gwait(buf, nsub):
        for i in range(NSUB):
            @pl.when(i < nsub)
            def _():
                pltpu.make_async_copy(
                    m_hbm.at[slist[0].at[pl.ds(0, SUB)]],
                    grows[buf].at[pl.ds(0, SUB)],
                    gsems[buf][i]).wait()

    @pl.loop(0, RPW + 1)
    def _(r):
        @pl.loop(0, W32, step=16)
        def _(k):
            acc[r, pl.ds(k, 16)] = zero16

    for p in range(2):
        @pl.loop(0, LCAP, step=16)
        def _(i):
            slist[p][pl.ds(i, 16)] = zero16

    def accum(buf, listpar, goff, gcnt):
        def group_body(g, carry):
            dlv = dlist[listpar][pl.ds((goff + g) * 16, 16)]
            for lane in range(16):
                dl = dlv[lane]
                e = g * 16 + lane
                for k in range(W32 // 16):
                    cur = plsc.bitcast(acc[dl, pl.ds(k * 16, 16)], jnp.bfloat16)
                    new = plsc.bitcast(grows[buf][e, pl.ds(k * 16, 16)],
                                       jnp.bfloat16)
                    acc[dl, pl.ds(k * 16, 16)] = plsc.bitcast(
                        jnp.maximum(cur, new), jnp.int32)
            return carry

        lax.fori_loop(0, gcnt, group_body, jnp.int32(0))

    def scan(par):
        def scan_body(i, ptr):
            base = i * 32
            d0 = dstv[par][pl.ds(base, 16)]
            s0 = srcv[par][pl.ds(base, 16)]
            d1 = dstv[par][pl.ds(base + 16, 16)]
            s1 = srcv[par][pl.ds(base + 16, 16)]
            r0 = d0 - lo
            r1 = d1 - lo
            m0 = r0.astype(jnp.uint32) < jnp.uint32(RPW)
            m1 = r1.astype(jnp.uint32) < jnp.uint32(RPW)
            plsc.store_compressed(slist[par].at[pl.ds(ptr, 16)], s0, mask=m0)
            plsc.store_compressed(dlist[par].at[pl.ds(ptr, 16)], r0, mask=m0)
            p1 = ptr + plsc.all_reduce_population_count(m0)[0]
            plsc.store_compressed(slist[par].at[pl.ds(p1, 16)], s1, mask=m1)
            plsc.store_compressed(dlist[par].at[pl.ds(p1, 16)], r1, mask=m1)
            return p1 + plsc.all_reduce_population_count(m1)[0]

        return lax.fori_loop(0, CHUNK // 32, scan_body, jnp.int32(0))

    def section(par, ci2):
        cur = ci2 * 2 + par

        def run(carry):
            prev_ng, prev_nsub = carry
            nxt = cur + 1
            estart(1 - par, jnp.where(nxt < NCHUNK, nxt, 0))
            ewait(par)
            ptr = scan(par)
            slist[par][pl.ds(ptr, 16)] = lanes
            dlist[par][pl.ds(ptr, 16)] = trash16
            ng_all = (ptr + 15) // 16
            ng = jnp.minimum(ng_all, NGMAX)
            nsub = (ng + (GPS - 1)) // GPS
            gstart(par, par, 0, nsub)
            gwait(1 - par, prev_nsub)
            accum(1 - par, 1 - par, jnp.int32(0), prev_ng)
            nxb = (ng_all + NGMAX - 1) // NGMAX

            @pl.when(nxb > 1)
            def _():
                def extra(b, carry2):
                    gstart(1 - par, par, b, jnp.int32(NSUB))
                    gwait(1 - par, jnp.int32(NSUB))
                    accum(1 - par, par, b * NGMAX,
                          jnp.minimum(ng_all - b * NGMAX, NGMAX))
                    return carry2
                lax.fori_loop(1, nxb, extra, jnp.int32(0))

            return (ng, nsub)
        return run

    def chunk_pair(ci2, carry):
        carry = section(0, ci2)(carry)
        carry = section(1, ci2)(carry)
        return carry

    estart(0, jnp.int32(0))
    carry = lax.fori_loop(0, NCHUNK // 2, chunk_pair,
                          (jnp.int32(0), jnp.int32(0)))
    prev_ng, prev_nsub = carry

    gwait(1, prev_nsub)
    accum(1, 1, jnp.int32(0), prev_ng)
    ewait(0)

    pltpu.sync_copy(acc.at[pl.ds(0, RPW)], agg_hbm.at[pl.ds(lo, RPW)])


def _seg_max(src, dst, m):
    mesh = plsc.VectorSubcoreMesh(core_axis_name="c", subcore_axis_name="s")
    cp = pltpu.CompilerParams()
    if "needs_layout_passes" in pltpu.CompilerParams.__dataclass_fields__:
        cp = dataclasses.replace(cp, needs_layout_passes=False)
    f = pl.kernel(
        _seg_max_body,
        out_type=jax.ShapeDtypeStruct((NPAD, W32), jnp.int32),
        mesh=mesh,
        compiler_params=cp,
        scratch_types=[
            pltpu.VMEM((CHUNK,), jnp.int32),
            pltpu.VMEM((CHUNK,), jnp.int32),
            pltpu.VMEM((CHUNK,), jnp.int32),
            pltpu.VMEM((CHUNK,), jnp.int32),
            pltpu.VMEM((LCAP,), jnp.int32),
            pltpu.VMEM((LCAP,), jnp.int32),
            pltpu.VMEM((LCAP,), jnp.int32),
            pltpu.VMEM((LCAP,), jnp.int32),
            pltpu.VMEM((GB, D), jnp.int32),
            pltpu.VMEM((GB, D), jnp.int32),
            pltpu.VMEM((RPW + 1, W32), jnp.int32),
        ] + [pltpu.SemaphoreType.DMA] * (2 + 2 * NSUB),
    )
    return f(src, dst, m)


def _dense_post_body(hws_ref, agg_ref, wn_ref, bs_ref, wf_ref, bf_ref,
                     g_ref, be_ref, z_ref):
    agg = agg_ref[...].astype(jnp.float32)
    rst = jax.nn.relu(
        hws_ref[...]
        + jnp.dot(agg, wn_ref[...], preferred_element_type=jnp.float32)
        + bs_ref[...]
    )
    nrm = jnp.maximum(
        jnp.sqrt(jnp.sum(rst * rst, axis=1, keepdims=True)), 1e-12)
    rst = rst / nrm
    z = jnp.dot(rst, wf_ref[...], preferred_element_type=jnp.float32) + bf_ref[...]
    z = z * (BN_SCALE * g_ref[...]) + be_ref[...]
    z_ref[...] = jax.nn.relu(z)


def _dense_post(hws, agg, W_neigh, b_sage2d, W_fc, b_fc2d, gamma2d, beta2d):
    blk = NPAD // 8
    return pl.pallas_call(
        _dense_post_body,
        grid=(8,),
        in_specs=[
            pl.BlockSpec((blk, D), lambda i: (i, 0)),
            pl.BlockSpec((blk, D), lambda i: (i, 0)),
            pl.BlockSpec((D, D), lambda i: (0, 0)),
            pl.BlockSpec((1, D), lambda i: (0, 0)),
            pl.BlockSpec((D, D), lambda i: (0, 0)),
            pl.BlockSpec((1, D), lambda i: (0, 0)),
            pl.BlockSpec((1, D), lambda i: (0, 0)),
            pl.BlockSpec((1, D), lambda i: (0, 0)),
        ],
        out_specs=pl.BlockSpec((blk, D), lambda i: (i, 0)),
        out_shape=jax.ShapeDtypeStruct((NPAD, D), jnp.float32),
    )(hws, agg, W_neigh, b_sage2d, W_fc, b_fc2d, gamma2d, beta2d)


def kernel(x, edge_index, W_pool, b_pool, W_self, W_neigh, b_sage,
           W_fc, b_fc, gamma, beta):
    xp = jnp.zeros((NPAD, D), jnp.float32).at[:N].set(x)
    m, hws = _dense_pre(xp, W_pool, b_pool.reshape(1, D), W_self)
    m32 = lax.bitcast_convert_type(m.reshape(NPAD, W32, 2), jnp.int32)
    m32 = jnp.pad(m32, ((0, 0), (0, D - W32)))
    agg32 = _seg_max(edge_index[0], edge_index[1], m32)
    agg = lax.bitcast_convert_type(agg32, jnp.bfloat16).reshape(NPAD, D)
    z = _dense_post(hws, agg, W_neigh, b_sage.reshape(1, D), W_fc,
                    b_fc.reshape(1, D), gamma.reshape(1, D), beta.reshape(1, D))
    return z[:N]


# speedup vs baseline: 7.3024x; 1.0266x over previous
import dataclasses
import functools

import jax
import jax.numpy as jnp
from jax import lax
from jax.experimental import pallas as pl
from jax.experimental.pallas import tpu as pltpu
from jax.experimental.pallas import tpu_sc as plsc

N = 10000
E = 320000
D = 128

NPAD = 10240
NW = 32
RPW = NPAD // NW
TRASH = RPW
CHUNK = 4000
NCHUNK = E // CHUNK
GB = 192
NGMAX = GB // 16
SUB = 16
NSUB = GB // SUB
GPS = SUB // 16
W32 = D // 2
LCAP = CHUNK + GB + 32
BN_SCALE = float(1.0 / (1.0 + 1e-5) ** 0.5)


def _dense_pre_body(x_ref, wp_ref, bp_ref, ws_ref, m_ref, hws_ref):
    h = jnp.log(x_ref[...] + 1.0)
    m = jax.nn.relu(
        jnp.dot(h, wp_ref[...], preferred_element_type=jnp.float32) + bp_ref[...]
    )
    m_ref[...] = m.astype(jnp.bfloat16)
    hws_ref[...] = jnp.dot(h, ws_ref[...], preferred_element_type=jnp.float32)


def _dense_pre(xp, W_pool, b_pool2d, W_self):
    blk = NPAD // 8
    return pl.pallas_call(
        _dense_pre_body,
        grid=(8,),
        in_specs=[
            pl.BlockSpec((blk, D), lambda i: (i, 0)),
            pl.BlockSpec((D, D), lambda i: (0, 0)),
            pl.BlockSpec((1, D), lambda i: (0, 0)),
            pl.BlockSpec((D, D), lambda i: (0, 0)),
        ],
        out_specs=[
            pl.BlockSpec((blk, D), lambda i: (i, 0)),
            pl.BlockSpec((blk, D), lambda i: (i, 0)),
        ],
        out_shape=[
            jax.ShapeDtypeStruct((NPAD, D), jnp.bfloat16),
            jax.ShapeDtypeStruct((NPAD, D), jnp.float32),
        ],
    )(xp, W_pool, b_pool2d, W_self)


def _seg_max_body(src_hbm, dst_hbm, m_hbm, agg_hbm,
                  srcv0, srcv1, dstv0, dstv1, sl0, sl1, dl0, dl1,
                  grows0, grows1, acc, se0, se1,
                  sg00, sg01, sg02, sg03, sg04, sg05,
                  sg06, sg07, sg08, sg09, sg0a, sg0b,
                  sg10, sg11, sg12, sg13, sg14, sg15,
                  sg16, sg17, sg18, sg19, sg1a, sg1b):
    srcv = (srcv0, srcv1)
    dstv = (dstv0, dstv1)
    slist = (sl0, sl1)
    dlist = (dl0, dl1)
    grows = (grows0, grows1)
    esems = (se0, se1)
    gsems = ((sg00, sg01, sg02, sg03, sg04, sg05,
              sg06, sg07, sg08, sg09, sg0a, sg0b),
             (sg10, sg11, sg12, sg13, sg14, sg15,
              sg16, sg17, sg18, sg19, sg1a, sg1b))

    wid = lax.axis_index("s") * 2 + lax.axis_index("c")
    lo = wid * RPW
    lanes = lax.broadcasted_iota(jnp.int32, (16,), 0)
    zero16 = jnp.zeros((16,), jnp.int32)
    trash16 = jnp.full((16,), TRASH, jnp.int32)

    def estart(par, ci):
        off = ci * CHUNK
        pltpu.make_async_copy(
            src_hbm.at[pl.ds(off, CHUNK)], srcv[par], esems[par]).start()
        pltpu.make_async_copy(
            dst_hbm.at[pl.ds(off, CHUNK)], dstv[par], esems[par]).start()

    def ewait(par):
        pltpu.make_async_copy(
            src_hbm.at[pl.ds(0, CHUNK)], srcv[par], esems[par]).wait()
        pltpu.make_async_copy(
            dst_hbm.at[pl.ds(0, CHUNK)], dstv[par], esems[par]).wait()

    def gstart(buf, listpar, b, nsub):
        for i in range(NSUB):
            @pl.when(i < nsub)
            def _():
                pltpu.make_async_copy(
                    m_hbm.at[slist[listpar].at[pl.ds(b * GB + i * SUB, SUB)]],
                    grows[buf].at[pl.ds(i * SUB, SUB)],
                    gsems[buf][i]).start()

    def gwait(buf, nsub):
        for i in range(NSUB):
            @pl.when(i < nsub)
            def _():
                pltpu.make_async_copy(
                    m_hbm.at[slist[0].at[pl.ds(0, SUB)]],
                    grows[buf].at[pl.ds(0, SUB)],
                    gsems[buf][i]).wait()

    @pl.loop(0, RPW + 1)
    def _(r):
        @pl.loop(0, W32, step=16)
        def _(k):
            acc[r, pl.ds(k, 16)] = zero16

    for p in range(2):
        @pl.loop(0, LCAP, step=16)
        def _(i):
            slist[p][pl.ds(i, 16)] = zero16

    def accum(buf, listpar, goff, gcnt):
        def group_body(g, carry):
            dlv = dlist[listpar][pl.ds((goff + g) * 16, 16)]
            for lane in range(16):
                dl = dlv[lane]
                e = g * 16 + lane
                for k in range(W32 // 16):
                    cur = plsc.bitcast(acc[dl, pl.ds(k * 16, 16)], jnp.bfloat16)
                    new = plsc.bitcast(grows[buf][e, pl.ds(k * 16, 16)],
                                       jnp.bfloat16)
                    acc[dl, pl.ds(k * 16, 16)] = plsc.bitcast(
                        jnp.maximum(cur, new), jnp.int32)
            return carry

        lax.fori_loop(0, gcnt, group_body, jnp.int32(0))

    def scan(par):
        def scan_body(i, ptr):
            base = i * 32
            d0 = dstv[par][pl.ds(base, 16)]
            s0 = srcv[par][pl.ds(base, 16)]
            d1 = dstv[par][pl.ds(base + 16, 16)]
            s1 = srcv[par][pl.ds(base + 16, 16)]
            r0 = d0 - lo
            r1 = d1 - lo
            m0 = r0.astype(jnp.uint32) < jnp.uint32(RPW)
            m1 = r1.astype(jnp.uint32) < jnp.uint32(RPW)
            plsc.store_compressed(slist[par].at[pl.ds(ptr, 16)], s0, mask=m0)
            plsc.store_compressed(dlist[par].at[pl.ds(ptr, 16)], r0, mask=m0)
            p1 = ptr + plsc.all_reduce_population_count(m0)[0]
            plsc.store_compressed(slist[par].at[pl.ds(p1, 16)], s1, mask=m1)
            plsc.store_compressed(dlist[par].at[pl.ds(p1, 16)], r1, mask=m1)
            return p1 + plsc.all_reduce_population_count(m1)[0]

        return lax.fori_loop(0, CHUNK // 32, scan_body, jnp.int32(0))

    def section(par, ci2):
        cur = ci2 * 2 + par

        def run(carry):
            prev_ng, prev_nsub = carry
            nxt = cur + 1
            estart(1 - par, jnp.where(nxt < NCHUNK, nxt, 0))
            ewait(par)
            ptr = scan(par)
            slist[par][pl.ds(ptr, 16)] = lanes
            dlist[par][pl.ds(ptr, 16)] = trash16
            ng_all = (ptr + 15) // 16
            ng = jnp.minimum(ng_all, NGMAX)
            nsub = (ng + (GPS - 1)) // GPS
            gstart(par, par, 0, nsub)
            gwait(1 - par, prev_nsub)
            accum(1 - par, 1 - par, jnp.int32(0), prev_ng)
            nxb = (ng_all + NGMAX - 1) // NGMAX

            @pl.when(nxb > 1)
            def _():
                def extra(b, carry2):
                    gstart(1 - par, par, b, jnp.int32(NSUB))
                    gwait(1 - par, jnp.int32(NSUB))
                    accum(1 - par, par, b * NGMAX,
                          jnp.minimum(ng_all - b * NGMAX, NGMAX))
                    return carry2
                lax.fori_loop(1, nxb, extra, jnp.int32(0))

            return (ng, nsub)
        return run

    def chunk_pair(ci2, carry):
        carry = section(0, ci2)(carry)
        carry = section(1, ci2)(carry)
        return carry

    estart(0, jnp.int32(0))
    carry = lax.fori_loop(0, NCHUNK // 2, chunk_pair,
                          (jnp.int32(0), jnp.int32(0)))
    prev_ng, prev_nsub = carry

    gwait(1, prev_nsub)
    accum(1, 1, jnp.int32(0), prev_ng)
    ewait(0)

    pltpu.sync_copy(acc.at[pl.ds(0, RPW)], agg_hbm.at[pl.ds(lo, RPW)])


def _seg_max(src, dst, m):
    mesh = plsc.VectorSubcoreMesh(core_axis_name="c", subcore_axis_name="s")
    cp = pltpu.CompilerParams()
    if "needs_layout_passes" in pltpu.CompilerParams.__dataclass_fields__:
        cp = dataclasses.replace(cp, needs_layout_passes=False)
    f = pl.kernel(
        _seg_max_body,
        out_type=jax.ShapeDtypeStruct((NPAD, W32), jnp.int32),
        mesh=mesh,
        compiler_params=cp,
        scratch_types=[
            pltpu.VMEM((CHUNK,), jnp.int32),
            pltpu.VMEM((CHUNK,), jnp.int32),
            pltpu.VMEM((CHUNK,), jnp.int32),
            pltpu.VMEM((CHUNK,), jnp.int32),
            pltpu.VMEM((LCAP,), jnp.int32),
            pltpu.VMEM((LCAP,), jnp.int32),
            pltpu.VMEM((LCAP,), jnp.int32),
            pltpu.VMEM((LCAP,), jnp.int32),
            pltpu.VMEM((GB, D), jnp.int32),
            pltpu.VMEM((GB, D), jnp.int32),
            pltpu.VMEM((RPW + 1, W32), jnp.int32),
        ] + [pltpu.SemaphoreType.DMA] * (2 + 2 * NSUB),
    )
    return f(src, dst, m)


def _dense_post_body(hws_ref, agg_ref, wn_ref, bs_ref, wf_ref, bf_ref,
                     g_ref, be_ref, z_ref):
    agg = agg_ref[...].astype(jnp.float32)
    rst = jax.nn.relu(
        hws_ref[...]
        + jnp.dot(agg, wn_ref[...], preferred_element_type=jnp.float32)
        + bs_ref[...]
    )
    nrm = jnp.maximum(
        jnp.sqrt(jnp.sum(rst * rst, axis=1, keepdims=True)), 1e-12)
    rst = rst / nrm
    z = jnp.dot(rst, wf_ref[...], preferred_element_type=jnp.float32) + bf_ref[...]
    z = z * (BN_SCALE * g_ref[...]) + be_ref[...]
    z_ref[...] = jax.nn.relu(z)


def _dense_post(hws, agg, W_neigh, b_sage2d, W_fc, b_fc2d, gamma2d, beta2d):
    blk = NPAD // 8
    return pl.pallas_call(
        _dense_post_body,
        grid=(8,),
        in_specs=[
            pl.BlockSpec((blk, D), lambda i: (i, 0)),
            pl.BlockSpec((blk, D), lambda i: (i, 0)),
            pl.BlockSpec((D, D), lambda i: (0, 0)),
            pl.BlockSpec((1, D), lambda i: (0, 0)),
            pl.BlockSpec((D, D), lambda i: (0, 0)),
            pl.BlockSpec((1, D), lambda i: (0, 0)),
            pl.BlockSpec((1, D), lambda i: (0, 0)),
            pl.BlockSpec((1, D), lambda i: (0, 0)),
        ],
        out_specs=pl.BlockSpec((blk, D), lambda i: (i, 0)),
        out_shape=jax.ShapeDtypeStruct((NPAD, D), jnp.float32),
    )(hws, agg, W_neigh, b_sage2d, W_fc, b_fc2d, gamma2d, beta2d)


def kernel(x, edge_index, W_pool, b_pool, W_self, W_neigh, b_sage,
           W_fc, b_fc, gamma, beta):
    xp = jnp.zeros((NPAD, D), jnp.float32).at[:N].set(x)
    m, hws = _dense_pre(xp, W_pool, b_pool.reshape(1, D), W_self)
    m32 = lax.bitcast_convert_type(m.reshape(NPAD, W32, 2), jnp.int32)
    m32 = jnp.pad(m32, ((0, 0), (0, D - W32)))
    agg32 = _seg_max(edge_index[0], edge_index[1], m32)
    agg = lax.bitcast_convert_type(agg32, jnp.bfloat16).reshape(NPAD, D)
    z = _dense_post(hws, agg, W_neigh, b_sage.reshape(1, D), W_fc,
                    b_fc.reshape(1, D), gamma.reshape(1, D), beta.reshape(1, D))
    return z[:N]
